# baseline (speedup 1.0000x reference)
import functools

import jax
import jax.numpy as jnp
from jax.experimental import pallas as pl
from jax.experimental.pallas import tpu as pltpu

S = 1024
H = 256


def _mv(v, W):
    return jax.lax.dot_general(
        v, W, (((1,), (1,)), ((), ())), preferred_element_type=jnp.float32
    )


def _kernel_body(x_ref, h_ref, bih_ref, bhh_ref, bout_ref,
                 wih_hbm, whh_hbm, wo_hbm, out_ref, hnew_ref,
                 wih_v, whh_v, wo_v, sem):
    c_ih = pltpu.make_async_copy(wih_hbm, wih_v, sem.at[0])
    c_hh = pltpu.make_async_copy(whh_hbm, whh_v, sem.at[1])
    c_o = [
        pltpu.make_async_copy(wo_hbm.at[q * H:(q + 1) * H, 0:3 * S],
                              wo_v.at[q * H:(q + 1) * H], sem.at[2 + q])
        for q in range(4)
    ]
    c_ih.start()
    c_hh.start()
    for c in c_o:
        c.start()

    x = x_ref[...]
    h = h_ref[...]

    c_ih.wait()
    gi = _mv(x, wih_v[...]) + bih_ref[...]
    c_hh.wait()
    gh = _mv(h, whh_v[...]) + bhh_ref[...]

    r = jax.nn.sigmoid(gi[:, :S] + gh[:, :S])
    z = jax.nn.sigmoid(gi[:, S:2 * S] + gh[:, S:2 * S])
    n = jnp.tanh(gi[:, 2 * S:] + r * gh[:, 2 * S:])
    hnew_ref[...] = ((1.0 - z) * n + z * h).reshape(1, 1, S)

    parts = []
    for q in range(4):
        c_o[q].wait()
        wq = wo_v[q * H:(q + 1) * H, :]
        parts.append(_mv(x, wq[:, :S]) + _mv(h, wq[:, S:2 * S])
                     + _mv(h, wq[:, 2 * S:]))
    logits = jnp.concatenate(parts, axis=1) + bout_ref[...]

    m = jnp.max(logits)
    lse = m + jnp.log(jnp.sum(jnp.exp(logits - m)))
    out_ref[...] = logits - lse


@functools.partial(jax.jit, static_argnames=())
def _run(feature, initial_h, W_ih, W_hh, b_ih, b_hh, W_out, b_out):
    x2 = feature.reshape(1, S)
    h2 = initial_h.reshape(1, S)
    bih = b_ih.reshape(1, 3 * S)
    bhh = b_hh.reshape(1, 3 * S)
    bout = b_out.reshape(1, S)

    vm = lambda: pl.BlockSpec(memory_space=pltpu.VMEM)
    anym = lambda: pl.BlockSpec(memory_space=pltpu.HBM)

    out, h_new = pl.pallas_call(
        _kernel_body,
        in_specs=[vm(), vm(), vm(), vm(), vm(), anym(), anym(), anym()],
        out_specs=[vm(), vm()],
        out_shape=[
            jax.ShapeDtypeStruct((1, S), jnp.float32),
            jax.ShapeDtypeStruct((1, 1, S), jnp.float32),
        ],
        scratch_shapes=[
            pltpu.VMEM((3 * S, S), jnp.float32),
            pltpu.VMEM((3 * S, S), jnp.float32),
            pltpu.VMEM((S, 3 * S), jnp.float32),
            pltpu.SemaphoreType.DMA((6,)),
        ],
    )(x2, h2, bih, bhh, bout, W_ih, W_hh, W_out)
    return out, h_new


def kernel(feature, time, initial_h, W_ih, W_hh, b_ih, b_hh, W_out, b_out):
    del time
    return _run(feature, initial_h, W_ih, W_hh, b_ih, b_hh, W_out, b_out)
